# baseline (speedup 1.0000x reference)
import functools

import jax
import jax.numpy as jnp
from jax.experimental import pallas as pl
from jax.experimental.pallas import tpu as pltpu


def _ru(x, m):
    return ((x + m - 1) // m) * m


def _proj_kernel(x_ref, w_ref, d_ref, z_ref):
    xb = x_ref[...].astype(jnp.bfloat16)
    z = jnp.dot(xb, w_ref[...], preferred_element_type=jnp.float32)
    z_ref[...] = z * d_ref[...]


def _gather_rows(cols_ref, src_ref, g_ref, ch, unroll):
    def body(o, _):
        base_t = o * unroll
        rows = []
        for ui in range(unroll):
            idx = cols_ref[0, 0, base_t + ui]
            rows.append(src_ref[pl.ds(idx, 1), :])
        for ui in range(unroll):
            g_ref[pl.ds(base_t + ui, 1), :] = rows[ui]
        return 0

    jax.lax.fori_loop(0, ch // unroll, body, 0, unroll=False)


def _deg_kernel(band_ref, lo_ref, hi_ref, first_ref, last_ref, chunk_ref,
                rows_ref, val_ref, o_ref, *, tm, ch, slots_half, num_nodes):
    g = pl.program_id(0)
    s = pl.program_id(1)
    sl = g * slots_half + s
    band = band_ref[sl]
    lo = lo_ref[sl]
    hi = hi_ref[sl]

    @pl.when(first_ref[sl] == 1)
    def _init():
        o_ref[...] = jnp.zeros_like(o_ref)

    @pl.when(hi > lo)
    def _accumulate():
        iota_t = jax.lax.broadcasted_iota(jnp.int32, (1, ch), 1)
        valm = jnp.where((iota_t >= lo) & (iota_t < hi), val_ref[0], 0.0)
        rl = rows_ref[0] - band * tm
        iota_r = jax.lax.broadcasted_iota(jnp.int32, (tm, ch), 0)
        oh = jnp.where(iota_r == rl, valm, 0.0)
        o_ref[...] += jnp.sum(oh, axis=1, keepdims=True)

    @pl.when(last_ref[sl] == 1)
    def _finalize():
        row_ids = band * tm + jax.lax.broadcasted_iota(
            jnp.int32, (tm, o_ref.shape[1]), 0)
        deg = o_ref[...] + (row_ids < num_nodes).astype(jnp.float32)
        o_ref[...] = jnp.where(deg > 0, jax.lax.rsqrt(deg), jnp.float32(0.0))


def _spmm_kernel(band_ref, lo_ref, hi_ref, first_ref, last_ref, chunk_ref,
                 cols_ref, rows_ref, val_ref, src_ref, d_ref, b_ref,
                 o_ref, g_ref, *, tm, ch, slots_half, unroll, final,
                 num_classes, c_pad):
    g = pl.program_id(0)
    s = pl.program_id(1)
    sl = g * slots_half + s
    band = band_ref[sl]
    lo = lo_ref[sl]
    hi = hi_ref[sl]

    @pl.when(first_ref[sl] == 1)
    def _init():
        o_ref[...] = src_ref[pl.ds(band * tm, tm), :]

    @pl.when(hi > lo)
    def _accumulate():
        _gather_rows(cols_ref, src_ref, g_ref, ch, unroll)
        iota_t = jax.lax.broadcasted_iota(jnp.int32, (1, ch), 1)
        valm = jnp.where((iota_t >= lo) & (iota_t < hi), val_ref[0], 0.0)
        rl = rows_ref[0] - band * tm
        iota_r = jax.lax.broadcasted_iota(jnp.int32, (tm, ch), 0)
        oh = jnp.where(iota_r == rl, valm, 0.0).astype(jnp.bfloat16)
        gb = g_ref[...].astype(jnp.bfloat16)
        o_ref[...] += jnp.dot(oh, gb, preferred_element_type=jnp.float32)

    if final:
        @pl.when(last_ref[sl] == 1)
        def _finalize():
            logits = o_ref[...] * d_ref[...] + b_ref[...]
            if num_classes < c_pad:
                col = jax.lax.broadcasted_iota(jnp.int32, logits.shape, 1)
                valid = col < num_classes
                logits = jnp.where(valid, logits, jnp.float32(-1e30))
                m = jnp.max(logits, axis=1, keepdims=True)
                e = jnp.where(valid, jnp.exp(logits - m), jnp.float32(0.0))
            else:
                m = jnp.max(logits, axis=1, keepdims=True)
                e = jnp.exp(logits - m)
            lse = jnp.log(jnp.sum(e, axis=1, keepdims=True)) + m
            o_ref[...] = logits - lse


def _slot_schedule(bstart, nb, nbh, core, nchunks, ch, slots_half):
    b0 = core * nbh
    nb_core = min(nbh, nb - b0)
    clo = bstart[b0:b0 + nb_core] // ch
    chi = jnp.maximum((bstart[b0 + 1:b0 + nb_core + 1] + ch - 1) // ch - 1, clo)
    span = chi - clo + 1
    scoff = jnp.concatenate([jnp.zeros(1, jnp.int32),
                             jnp.cumsum(span).astype(jnp.int32)])
    s = jnp.arange(slots_half, dtype=jnp.int32)
    bloc = jnp.clip(jnp.sum((scoff[None, :] <= s[:, None]).astype(jnp.int32),
                            axis=1) - 1, 0, nb_core - 1)
    band = (b0 + bloc).astype(jnp.int32)
    within = s - scoff[bloc]
    chunk = jnp.clip(clo[bloc] + within, 0, nchunks - 1).astype(jnp.int32)
    real = s < scoff[nb_core]
    base = chunk * ch
    lo = jnp.where(real, jnp.maximum(bstart[band], base) - base, 0)
    hi = jnp.where(real, jnp.minimum(bstart[band + 1], base + ch) - base, 0)
    hi = jnp.maximum(hi, lo)
    first = ((s == scoff[bloc]) & real).astype(jnp.int32)
    last = ((s == scoff[bloc + 1] - 1) & real).astype(jnp.int32)
    return band, lo.astype(jnp.int32), hi.astype(jnp.int32), first, last, chunk


def kernel(edge_index, x, weight, bias):
    N, F = x.shape
    C = weight.shape[1]
    tm = 512
    ch = 1024
    unroll = 16
    n_pad = _ru(N, 2 * tm)
    f_pad = _ru(F, 128)
    c_pad = _ru(C, 128)
    nb = n_pad // tm
    nbh = nb // 2

    src, dst = edge_index[0], edge_index[1]
    rows = jnp.concatenate([src, dst])
    cols = jnp.concatenate([dst, src])
    m = rows.shape[0]
    m_pad = _ru(m, ch)
    key = jnp.sort(rows * n_pad + cols)
    pad_key = jnp.full((m_pad - m,), n_pad * n_pad - 1, jnp.int32)
    key = jnp.concatenate([key, pad_key])
    rs = (key // n_pad).astype(jnp.int32)
    cs = (key % n_pad).astype(jnp.int32)
    is_real = jnp.arange(m_pad) < m
    uniq = jnp.concatenate([jnp.ones(1, jnp.bool_), key[1:] != key[:-1]])
    uniq = uniq & is_real
    val = uniq.astype(jnp.float32)

    nchunks = m_pad // ch
    slots_half = nchunks + nbh + 1
    q = (jnp.arange(nb + 1, dtype=jnp.int32) * tm)[:, None]
    bstart = jnp.sum((rs[None, :] < q).astype(jnp.int32), axis=1)
    sched = [jnp.concatenate([a, b]) for a, b in zip(
        _slot_schedule(bstart, nb, nbh, 0, nchunks, ch, slots_half),
        _slot_schedule(bstart, nb, nbh, 1, nchunks, ch, slots_half))]
    band_s, lo_s, hi_s, first_s, last_s, chunk_s = sched

    rows3 = rs.reshape(nchunks, 1, ch)
    cols3 = cs.reshape(nchunks, 1, ch)
    val3 = val.reshape(nchunks, 1, ch)

    def edge_map3(g, s, *pref):
        return (pref[5][g * slots_half + s], 0, 0)

    def out_map2(g, s, *pref):
        return (pref[0][g * slots_half + s], 0)

    d1 = pl.pallas_call(
        functools.partial(_deg_kernel, tm=tm, ch=ch, slots_half=slots_half,
                          num_nodes=N),
        out_shape=jax.ShapeDtypeStruct((n_pad, c_pad), jnp.float32),
        grid_spec=pltpu.PrefetchScalarGridSpec(
            num_scalar_prefetch=6,
            grid=(2, slots_half),
            in_specs=[
                pl.BlockSpec((1, 1, ch), edge_map3),
                pl.BlockSpec((1, 1, ch), edge_map3),
            ],
            out_specs=pl.BlockSpec((tm, c_pad), out_map2)),
        compiler_params=pltpu.CompilerParams(
            dimension_semantics=("parallel", "arbitrary")),
    )(band_s, lo_s, hi_s, first_s, last_s, chunk_s, rows3, val3)

    if N == n_pad and F == f_pad:
        x_p = x
    else:
        x_p = jnp.zeros((n_pad, f_pad), x.dtype).at[:N, :F].set(x)
    w_p = jnp.zeros((f_pad, c_pad), jnp.bfloat16).at[:F, :C].set(
        weight.astype(jnp.bfloat16))
    b_p = jnp.zeros((1, c_pad), jnp.float32).at[0, :C].set(
        bias.astype(jnp.float32))

    z0 = pl.pallas_call(
        _proj_kernel,
        out_shape=jax.ShapeDtypeStruct((n_pad, c_pad), jnp.float32),
        grid=(nb,),
        in_specs=[
            pl.BlockSpec((tm, f_pad), lambda i: (i, 0)),
            pl.BlockSpec((f_pad, c_pad), lambda i: (0, 0)),
            pl.BlockSpec((tm, c_pad), lambda i: (i, 0)),
        ],
        out_specs=pl.BlockSpec((tm, c_pad), lambda i: (i, 0)),
        compiler_params=pltpu.CompilerParams(
            dimension_semantics=("parallel",)),
    )(x_p, w_p, d1)

    def spmm(source, final):
        kern = functools.partial(
            _spmm_kernel, tm=tm, ch=ch, slots_half=slots_half, unroll=unroll,
            final=final, num_classes=C, c_pad=c_pad)
        smem = pltpu.MemorySpace.SMEM
        nargs = 6

        def edge_map(g, s, *pref):
            return (pref[5][g * slots_half + s], 0, 0)

        def out_map(g, s, *pref):
            return (pref[0][g * slots_half + s], 0)

        return pl.pallas_call(
            kern,
            out_shape=jax.ShapeDtypeStruct((n_pad, c_pad), jnp.float32),
            grid_spec=pltpu.PrefetchScalarGridSpec(
                num_scalar_prefetch=nargs,
                grid=(2, slots_half),
                in_specs=[
                    pl.BlockSpec((1, 1, ch), edge_map, memory_space=smem),
                    pl.BlockSpec((1, 1, ch), edge_map),
                    pl.BlockSpec((1, 1, ch), edge_map),
                    pl.BlockSpec((n_pad, c_pad), lambda g, s, *p: (0, 0)),
                    pl.BlockSpec((tm, c_pad), out_map),
                    pl.BlockSpec((1, c_pad), lambda g, s, *p: (0, 0)),
                ],
                out_specs=pl.BlockSpec((tm, c_pad), out_map),
                scratch_shapes=[pltpu.VMEM((ch, c_pad), jnp.float32)]),
            compiler_params=pltpu.CompilerParams(
                dimension_semantics=("parallel", "arbitrary"),
                vmem_limit_bytes=100 * 1024 * 1024),
        )(band_s, lo_s, hi_s, first_s, last_s, chunk_s,
          cols3, rows3, val3, source, d1, b_p)

    h = spmm(z0, final=False)
    h = h * d1 * d1

    out_p = spmm(h, final=True)

    return out_p[:N, :C]


# speedup vs baseline: 1.1810x; 1.1810x over previous
import functools

import jax
import jax.numpy as jnp
from jax.experimental import pallas as pl
from jax.experimental.pallas import tpu as pltpu


def _ru(x, m):
    return ((x + m - 1) // m) * m


def _proj_kernel(x_ref, w_ref, d_ref, z_ref):
    xb = x_ref[...].astype(jnp.bfloat16)
    z = jnp.dot(xb, w_ref[...], preferred_element_type=jnp.float32)
    z_ref[...] = z * d_ref[...]


def _gather_rows(cols_ref, src_ref, g_ref, ch, unroll):
    def body(o, _):
        base_t = o * unroll
        rows = []
        for ui in range(unroll):
            idx = cols_ref[0, 0, base_t + ui]
            rows.append(src_ref[pl.ds(idx, 1), :])
        for ui in range(unroll):
            g_ref[pl.ds(base_t + ui, 1), :] = rows[ui]
        return 0

    jax.lax.fori_loop(0, ch // unroll, body, 0, unroll=False)


def _deg_kernel(band_ref, lo_ref, hi_ref, first_ref, last_ref, chunk_ref,
                rows_ref, val_ref, o_ref, *, tm, ch, slots_half, num_nodes):
    g = pl.program_id(0)
    s = pl.program_id(1)
    sl = g * slots_half + s
    band = band_ref[sl]
    lo = lo_ref[sl]
    hi = hi_ref[sl]

    @pl.when(first_ref[sl] == 1)
    def _init():
        o_ref[...] = jnp.zeros_like(o_ref)

    @pl.when(hi > lo)
    def _accumulate():
        iota_t = jax.lax.broadcasted_iota(jnp.int32, (1, ch), 1)
        valm = jnp.where((iota_t >= lo) & (iota_t < hi), val_ref[0], 0.0)
        rl = rows_ref[0] - band * tm
        iota_r = jax.lax.broadcasted_iota(jnp.int32, (tm, ch), 0)
        oh = jnp.where(iota_r == rl, valm, 0.0)
        o_ref[...] += jnp.sum(oh, axis=1, keepdims=True)

    @pl.when(last_ref[sl] == 1)
    def _finalize():
        row_ids = band * tm + jax.lax.broadcasted_iota(
            jnp.int32, (tm, o_ref.shape[1]), 0)
        deg = o_ref[...] + (row_ids < num_nodes).astype(jnp.float32)
        o_ref[...] = jnp.where(deg > 0, jax.lax.rsqrt(deg), jnp.float32(0.0))


def _spmm_kernel(band_ref, lo_ref, hi_ref, first_ref, last_ref, chunk_ref,
                 cols_ref, rows_ref, val_ref, src_ref, d_ref, b_ref,
                 o_ref, g_ref, *, tm, ch, slots_half, unroll, final,
                 num_classes, c_pad):
    g = pl.program_id(0)
    s = pl.program_id(1)
    sl = g * slots_half + s
    band = band_ref[sl]
    lo = lo_ref[sl]
    hi = hi_ref[sl]

    @pl.when(first_ref[sl] == 1)
    def _init():
        o_ref[...] = src_ref[pl.ds(band * tm, tm), :]

    @pl.when(hi > lo)
    def _accumulate():
        _gather_rows(cols_ref, src_ref, g_ref, ch, unroll)
        iota_t = jax.lax.broadcasted_iota(jnp.int32, (1, ch), 1)
        valm = jnp.where((iota_t >= lo) & (iota_t < hi), val_ref[0], 0.0)
        rl = rows_ref[0] - band * tm
        iota_r = jax.lax.broadcasted_iota(jnp.int32, (tm, ch), 0)
        oh = jnp.where(iota_r == rl, valm, 0.0).astype(jnp.bfloat16)
        gb = g_ref[...].astype(jnp.bfloat16)
        o_ref[...] += jnp.dot(oh, gb, preferred_element_type=jnp.float32)

    if final:
        @pl.when(last_ref[sl] == 1)
        def _finalize():
            logits = o_ref[...] * d_ref[...] + b_ref[...]
            if num_classes < c_pad:
                col = jax.lax.broadcasted_iota(jnp.int32, logits.shape, 1)
                valid = col < num_classes
                logits = jnp.where(valid, logits, jnp.float32(-1e30))
                m = jnp.max(logits, axis=1, keepdims=True)
                e = jnp.where(valid, jnp.exp(logits - m), jnp.float32(0.0))
            else:
                m = jnp.max(logits, axis=1, keepdims=True)
                e = jnp.exp(logits - m)
            lse = jnp.log(jnp.sum(e, axis=1, keepdims=True)) + m
            o_ref[...] = logits - lse


def _slot_schedule(bstart, nb, nbh, core, nchunks, ch, slots_half):
    b0 = core * nbh
    nb_core = min(nbh, nb - b0)
    clo = bstart[b0:b0 + nb_core] // ch
    chi = jnp.maximum((bstart[b0 + 1:b0 + nb_core + 1] + ch - 1) // ch - 1, clo)
    span = chi - clo + 1
    scoff = jnp.concatenate([jnp.zeros(1, jnp.int32),
                             jnp.cumsum(span).astype(jnp.int32)])
    s = jnp.arange(slots_half, dtype=jnp.int32)
    bloc = jnp.clip(jnp.sum((scoff[None, :] <= s[:, None]).astype(jnp.int32),
                            axis=1) - 1, 0, nb_core - 1)
    band = (b0 + bloc).astype(jnp.int32)
    within = s - scoff[bloc]
    chunk = jnp.clip(clo[bloc] + within, 0, nchunks - 1).astype(jnp.int32)
    real = s < scoff[nb_core]
    base = chunk * ch
    lo = jnp.where(real, jnp.maximum(bstart[band], base) - base, 0)
    hi = jnp.where(real, jnp.minimum(bstart[band + 1], base + ch) - base, 0)
    hi = jnp.maximum(hi, lo)
    first = ((s == scoff[bloc]) & real).astype(jnp.int32)
    last = ((s == scoff[bloc + 1] - 1) & real).astype(jnp.int32)
    return band, lo.astype(jnp.int32), hi.astype(jnp.int32), first, last, chunk


def kernel(edge_index, x, weight, bias):
    N, F = x.shape
    C = weight.shape[1]
    tm = 512
    ch = 2048
    unroll = 32
    n_pad = _ru(N, 2 * tm)
    f_pad = _ru(F, 128)
    c_pad = _ru(C, 128)
    nb = n_pad // tm
    nbh = nb // 2

    src, dst = edge_index[0], edge_index[1]
    rows = jnp.concatenate([src, dst])
    cols = jnp.concatenate([dst, src])
    m = rows.shape[0]
    m_pad = _ru(m, ch)
    key = jnp.sort(rows * n_pad + cols)
    pad_key = jnp.full((m_pad - m,), n_pad * n_pad - 1, jnp.int32)
    key = jnp.concatenate([key, pad_key])
    rs = (key // n_pad).astype(jnp.int32)
    cs = (key % n_pad).astype(jnp.int32)
    is_real = jnp.arange(m_pad) < m
    uniq = jnp.concatenate([jnp.ones(1, jnp.bool_), key[1:] != key[:-1]])
    uniq = uniq & is_real
    val = uniq.astype(jnp.float32)

    nchunks = m_pad // ch
    slots_half = nchunks + nbh + 1
    q = (jnp.arange(nb + 1, dtype=jnp.int32) * tm)[:, None]
    bstart = jnp.sum((rs[None, :] < q).astype(jnp.int32), axis=1)
    sched = [jnp.concatenate([a, b]) for a, b in zip(
        _slot_schedule(bstart, nb, nbh, 0, nchunks, ch, slots_half),
        _slot_schedule(bstart, nb, nbh, 1, nchunks, ch, slots_half))]
    band_s, lo_s, hi_s, first_s, last_s, chunk_s = sched

    rows3 = rs.reshape(nchunks, 1, ch)
    cols3 = cs.reshape(nchunks, 1, ch)
    val3 = val.reshape(nchunks, 1, ch)

    def edge_map3(g, s, *pref):
        return (pref[5][g * slots_half + s], 0, 0)

    def out_map2(g, s, *pref):
        return (pref[0][g * slots_half + s], 0)

    d1 = pl.pallas_call(
        functools.partial(_deg_kernel, tm=tm, ch=ch, slots_half=slots_half,
                          num_nodes=N),
        out_shape=jax.ShapeDtypeStruct((n_pad, c_pad), jnp.float32),
        grid_spec=pltpu.PrefetchScalarGridSpec(
            num_scalar_prefetch=6,
            grid=(2, slots_half),
            in_specs=[
                pl.BlockSpec((1, 1, ch), edge_map3),
                pl.BlockSpec((1, 1, ch), edge_map3),
            ],
            out_specs=pl.BlockSpec((tm, c_pad), out_map2)),
        compiler_params=pltpu.CompilerParams(
            dimension_semantics=("parallel", "arbitrary")),
    )(band_s, lo_s, hi_s, first_s, last_s, chunk_s, rows3, val3)

    if N == n_pad and F == f_pad:
        x_p = x
    else:
        x_p = jnp.zeros((n_pad, f_pad), x.dtype).at[:N, :F].set(x)
    w_p = jnp.zeros((f_pad, c_pad), jnp.bfloat16).at[:F, :C].set(
        weight.astype(jnp.bfloat16))
    b_p = jnp.zeros((1, c_pad), jnp.float32).at[0, :C].set(
        bias.astype(jnp.float32))

    z0 = pl.pallas_call(
        _proj_kernel,
        out_shape=jax.ShapeDtypeStruct((n_pad, c_pad), jnp.float32),
        grid=(nb,),
        in_specs=[
            pl.BlockSpec((tm, f_pad), lambda i: (i, 0)),
            pl.BlockSpec((f_pad, c_pad), lambda i: (0, 0)),
            pl.BlockSpec((tm, c_pad), lambda i: (i, 0)),
        ],
        out_specs=pl.BlockSpec((tm, c_pad), lambda i: (i, 0)),
        compiler_params=pltpu.CompilerParams(
            dimension_semantics=("parallel",)),
    )(x_p, w_p, d1)

    def spmm(source, final):
        kern = functools.partial(
            _spmm_kernel, tm=tm, ch=ch, slots_half=slots_half, unroll=unroll,
            final=final, num_classes=C, c_pad=c_pad)
        smem = pltpu.MemorySpace.SMEM
        nargs = 6

        def edge_map(g, s, *pref):
            return (pref[5][g * slots_half + s], 0, 0)

        def out_map(g, s, *pref):
            return (pref[0][g * slots_half + s], 0)

        return pl.pallas_call(
            kern,
            out_shape=jax.ShapeDtypeStruct((n_pad, c_pad), jnp.float32),
            grid_spec=pltpu.PrefetchScalarGridSpec(
                num_scalar_prefetch=nargs,
                grid=(2, slots_half),
                in_specs=[
                    pl.BlockSpec((1, 1, ch), edge_map, memory_space=smem),
                    pl.BlockSpec((1, 1, ch), edge_map),
                    pl.BlockSpec((1, 1, ch), edge_map),
                    pl.BlockSpec((n_pad, c_pad), lambda g, s, *p: (0, 0)),
                    pl.BlockSpec((tm, c_pad), out_map),
                    pl.BlockSpec((1, c_pad), lambda g, s, *p: (0, 0)),
                ],
                out_specs=pl.BlockSpec((tm, c_pad), out_map),
                scratch_shapes=[pltpu.VMEM((ch, c_pad), jnp.float32)]),
            compiler_params=pltpu.CompilerParams(
                dimension_semantics=("parallel", "arbitrary"),
                vmem_limit_bytes=100 * 1024 * 1024),
        )(band_s, lo_s, hi_s, first_s, last_s, chunk_s,
          cols3, rows3, val3, source, d1, b_p)

    h = spmm(z0, final=False)
    h = h * d1 * d1

    out_p = spmm(h, final=True)

    return out_p[:N, :C]


# speedup vs baseline: 1.2173x; 1.0308x over previous
import functools

import jax
import jax.numpy as jnp
from jax.experimental import pallas as pl
from jax.experimental.pallas import tpu as pltpu


def _ru(x, m):
    return ((x + m - 1) // m) * m


def _proj_kernel(x_ref, w_ref, d_ref, z_ref):
    xb = x_ref[...].astype(jnp.bfloat16)
    z = jnp.dot(xb, w_ref[...], preferred_element_type=jnp.float32)
    z_ref[...] = z * d_ref[...]


def _gather_rows(cols_ref, src_ref, g_ref, ch, unroll):
    def body(o, _):
        base_t = o * unroll
        rows = []
        for ui in range(unroll):
            idx = cols_ref[0, 0, base_t + ui]
            rows.append(src_ref[pl.ds(idx, 1), :])
        for ui in range(unroll):
            g_ref[pl.ds(base_t + ui, 1), :] = rows[ui]
        return 0

    jax.lax.fori_loop(0, ch // unroll, body, 0, unroll=False)


def _deg_kernel(band_ref, lo_ref, hi_ref, first_ref, last_ref, chunk_ref,
                rows_ref, val_ref, o_ref, *, tm, ch, slots_half, num_nodes):
    g = pl.program_id(0)
    s = pl.program_id(1)
    sl = g * slots_half + s
    band = band_ref[sl]
    lo = lo_ref[sl]
    hi = hi_ref[sl]

    @pl.when(first_ref[sl] == 1)
    def _init():
        o_ref[...] = jnp.zeros_like(o_ref)

    @pl.when(hi > lo)
    def _accumulate():
        iota_t = jax.lax.broadcasted_iota(jnp.int32, (1, ch), 1)
        valm = jnp.where((iota_t >= lo) & (iota_t < hi), val_ref[0], 0.0)
        rl = rows_ref[0] - band * tm
        iota_r = jax.lax.broadcasted_iota(jnp.int32, (tm, ch), 0)
        oh = jnp.where(iota_r == rl, valm, 0.0)
        o_ref[...] += jnp.sum(oh, axis=1, keepdims=True)

    @pl.when(last_ref[sl] == 1)
    def _finalize():
        row_ids = band * tm + jax.lax.broadcasted_iota(
            jnp.int32, (tm, o_ref.shape[1]), 0)
        deg = o_ref[...] + (row_ids < num_nodes).astype(jnp.float32)
        o_ref[...] = jnp.where(deg > 0, jax.lax.rsqrt(deg), jnp.float32(0.0))


def _spmm_kernel(band_ref, lo_ref, hi_ref, first_ref, last_ref, chunk_ref,
                 cols_ref, rows_ref, val_ref, src_ref, d_ref, b_ref,
                 o_ref, g_ref, *, tm, ch, slots_half, unroll, final,
                 num_classes, c_pad):
    g = pl.program_id(0)
    s = pl.program_id(1)
    sl = g * slots_half + s
    band = band_ref[sl]
    lo = lo_ref[sl]
    hi = hi_ref[sl]

    @pl.when(first_ref[sl] == 1)
    def _init():
        o_ref[...] = src_ref[pl.ds(band * tm, tm), :]

    @pl.when(hi > lo)
    def _accumulate():
        _gather_rows(cols_ref, src_ref, g_ref, ch, unroll)
        iota_t = jax.lax.broadcasted_iota(jnp.int32, (1, ch), 1)
        valm = jnp.where((iota_t >= lo) & (iota_t < hi), val_ref[0], 0.0)
        rl = rows_ref[0] - band * tm
        iota_r = jax.lax.broadcasted_iota(jnp.int32, (tm, ch), 0)
        oh = jnp.where(iota_r == rl, valm, 0.0).astype(jnp.bfloat16)
        gb = g_ref[...].astype(jnp.bfloat16)
        o_ref[...] += jnp.dot(oh, gb, preferred_element_type=jnp.float32)

    if final:
        @pl.when(last_ref[sl] == 1)
        def _finalize():
            logits = o_ref[...] * d_ref[...] + b_ref[...]
            if num_classes < c_pad:
                col = jax.lax.broadcasted_iota(jnp.int32, logits.shape, 1)
                valid = col < num_classes
                logits = jnp.where(valid, logits, jnp.float32(-1e30))
                m = jnp.max(logits, axis=1, keepdims=True)
                e = jnp.where(valid, jnp.exp(logits - m), jnp.float32(0.0))
            else:
                m = jnp.max(logits, axis=1, keepdims=True)
                e = jnp.exp(logits - m)
            lse = jnp.log(jnp.sum(e, axis=1, keepdims=True)) + m
            o_ref[...] = logits - lse


def _slot_schedule(bstart, nb, nbh, core, nchunks, ch, slots_half):
    b0 = core * nbh
    nb_core = min(nbh, nb - b0)
    clo = bstart[b0:b0 + nb_core] // ch
    chi = jnp.maximum((bstart[b0 + 1:b0 + nb_core + 1] + ch - 1) // ch - 1, clo)
    span = chi - clo + 1
    scoff = jnp.concatenate([jnp.zeros(1, jnp.int32),
                             jnp.cumsum(span).astype(jnp.int32)])
    s = jnp.arange(slots_half, dtype=jnp.int32)
    bloc = jnp.clip(jnp.sum((scoff[None, :] <= s[:, None]).astype(jnp.int32),
                            axis=1) - 1, 0, nb_core - 1)
    band = (b0 + bloc).astype(jnp.int32)
    within = s - scoff[bloc]
    chunk = jnp.clip(clo[bloc] + within, 0, nchunks - 1).astype(jnp.int32)
    real = s < scoff[nb_core]
    base = chunk * ch
    lo = jnp.where(real, jnp.maximum(bstart[band], base) - base, 0)
    hi = jnp.where(real, jnp.minimum(bstart[band + 1], base + ch) - base, 0)
    hi = jnp.maximum(hi, lo)
    first = ((s == scoff[bloc]) & real).astype(jnp.int32)
    last = ((s == scoff[bloc + 1] - 1) & real).astype(jnp.int32)
    return band, lo.astype(jnp.int32), hi.astype(jnp.int32), first, last, chunk


def kernel(edge_index, x, weight, bias):
    N, F = x.shape
    C = weight.shape[1]
    tm = 512
    ch = 2048
    unroll = 64
    n_pad = _ru(N, 2 * tm)
    f_pad = _ru(F, 128)
    c_pad = _ru(C, 128)
    nb = n_pad // tm
    nbh = nb // 2

    src, dst = edge_index[0], edge_index[1]
    rows = jnp.concatenate([src, dst])
    cols = jnp.concatenate([dst, src])
    m = rows.shape[0]
    m_pad = _ru(m, ch)
    key = jnp.sort(rows * n_pad + cols)
    pad_key = jnp.full((m_pad - m,), n_pad * n_pad - 1, jnp.int32)
    key = jnp.concatenate([key, pad_key])
    rs = (key // n_pad).astype(jnp.int32)
    cs = (key % n_pad).astype(jnp.int32)
    is_real = jnp.arange(m_pad) < m
    uniq = jnp.concatenate([jnp.ones(1, jnp.bool_), key[1:] != key[:-1]])
    uniq = uniq & is_real
    val = uniq.astype(jnp.float32)

    nchunks = m_pad // ch
    slots_half = nchunks + nbh + 1
    q = (jnp.arange(nb + 1, dtype=jnp.int32) * tm)[:, None]
    bstart = jnp.sum((rs[None, :] < q).astype(jnp.int32), axis=1)
    sched = [jnp.concatenate([a, b]) for a, b in zip(
        _slot_schedule(bstart, nb, nbh, 0, nchunks, ch, slots_half),
        _slot_schedule(bstart, nb, nbh, 1, nchunks, ch, slots_half))]
    band_s, lo_s, hi_s, first_s, last_s, chunk_s = sched

    rows3 = rs.reshape(nchunks, 1, ch)
    cols3 = cs.reshape(nchunks, 1, ch)
    val3 = val.reshape(nchunks, 1, ch)

    def edge_map3(g, s, *pref):
        return (pref[5][g * slots_half + s], 0, 0)

    def out_map2(g, s, *pref):
        return (pref[0][g * slots_half + s], 0)

    d1 = pl.pallas_call(
        functools.partial(_deg_kernel, tm=tm, ch=ch, slots_half=slots_half,
                          num_nodes=N),
        out_shape=jax.ShapeDtypeStruct((n_pad, c_pad), jnp.float32),
        grid_spec=pltpu.PrefetchScalarGridSpec(
            num_scalar_prefetch=6,
            grid=(2, slots_half),
            in_specs=[
                pl.BlockSpec((1, 1, ch), edge_map3),
                pl.BlockSpec((1, 1, ch), edge_map3),
            ],
            out_specs=pl.BlockSpec((tm, c_pad), out_map2)),
        compiler_params=pltpu.CompilerParams(
            dimension_semantics=("parallel", "arbitrary")),
    )(band_s, lo_s, hi_s, first_s, last_s, chunk_s, rows3, val3)

    if N == n_pad and F == f_pad:
        x_p = x
    else:
        x_p = jnp.zeros((n_pad, f_pad), x.dtype).at[:N, :F].set(x)
    w_p = jnp.zeros((f_pad, c_pad), jnp.bfloat16).at[:F, :C].set(
        weight.astype(jnp.bfloat16))
    b_p = jnp.zeros((1, c_pad), jnp.float32).at[0, :C].set(
        bias.astype(jnp.float32))

    z0 = pl.pallas_call(
        _proj_kernel,
        out_shape=jax.ShapeDtypeStruct((n_pad, c_pad), jnp.float32),
        grid=(nb,),
        in_specs=[
            pl.BlockSpec((tm, f_pad), lambda i: (i, 0)),
            pl.BlockSpec((f_pad, c_pad), lambda i: (0, 0)),
            pl.BlockSpec((tm, c_pad), lambda i: (i, 0)),
        ],
        out_specs=pl.BlockSpec((tm, c_pad), lambda i: (i, 0)),
        compiler_params=pltpu.CompilerParams(
            dimension_semantics=("parallel",)),
    )(x_p, w_p, d1)

    def spmm(source, final):
        kern = functools.partial(
            _spmm_kernel, tm=tm, ch=ch, slots_half=slots_half, unroll=unroll,
            final=final, num_classes=C, c_pad=c_pad)
        smem = pltpu.MemorySpace.SMEM
        nargs = 6

        def edge_map(g, s, *pref):
            return (pref[5][g * slots_half + s], 0, 0)

        def out_map(g, s, *pref):
            return (pref[0][g * slots_half + s], 0)

        return pl.pallas_call(
            kern,
            out_shape=jax.ShapeDtypeStruct((n_pad, c_pad), jnp.float32),
            grid_spec=pltpu.PrefetchScalarGridSpec(
                num_scalar_prefetch=nargs,
                grid=(2, slots_half),
                in_specs=[
                    pl.BlockSpec((1, 1, ch), edge_map, memory_space=smem),
                    pl.BlockSpec((1, 1, ch), edge_map),
                    pl.BlockSpec((1, 1, ch), edge_map),
                    pl.BlockSpec((n_pad, c_pad), lambda g, s, *p: (0, 0)),
                    pl.BlockSpec((tm, c_pad), out_map),
                    pl.BlockSpec((1, c_pad), lambda g, s, *p: (0, 0)),
                ],
                out_specs=pl.BlockSpec((tm, c_pad), out_map),
                scratch_shapes=[pltpu.VMEM((ch, c_pad), jnp.float32)]),
            compiler_params=pltpu.CompilerParams(
                dimension_semantics=("parallel", "arbitrary"),
                vmem_limit_bytes=100 * 1024 * 1024),
        )(band_s, lo_s, hi_s, first_s, last_s, chunk_s,
          cols3, rows3, val3, source, d1, b_p)

    h = spmm(z0, final=False)
    h = h * d1 * d1

    out_p = spmm(h, final=True)

    return out_p[:N, :C]


# speedup vs baseline: 1.2366x; 1.0158x over previous
import functools

import jax
import jax.numpy as jnp
from jax.experimental import pallas as pl
from jax.experimental.pallas import tpu as pltpu


def _ru(x, m):
    return ((x + m - 1) // m) * m


def _proj_kernel(x_ref, w_ref, d_ref, z_ref):
    xb = x_ref[...].astype(jnp.bfloat16)
    z = jnp.dot(xb, w_ref[...], preferred_element_type=jnp.float32)
    z_ref[...] = z * d_ref[...]


def _gather_rows(cols_ref, src_ref, g_ref, ch, unroll):
    def body(o, _):
        base_t = o * unroll
        rows = []
        for ui in range(unroll):
            idx = cols_ref[0, 0, base_t + ui]
            rows.append(src_ref[pl.ds(idx, 1), :])
        for ui in range(unroll):
            g_ref[pl.ds(base_t + ui, 1), :] = rows[ui]
        return 0

    jax.lax.fori_loop(0, ch // unroll, body, 0, unroll=False)


def _deg_kernel(band_ref, lo_ref, hi_ref, first_ref, last_ref, chunk_ref,
                rows_ref, val_ref, o_ref, *, tm, ch, slots_half, num_nodes):
    g = pl.program_id(0)
    s = pl.program_id(1)
    sl = g * slots_half + s
    band = band_ref[sl]
    lo = lo_ref[sl]
    hi = hi_ref[sl]

    @pl.when(first_ref[sl] == 1)
    def _init():
        o_ref[...] = jnp.zeros_like(o_ref)

    @pl.when(hi > lo)
    def _accumulate():
        iota_t = jax.lax.broadcasted_iota(jnp.int32, (1, ch), 1)
        valm = jnp.where((iota_t >= lo) & (iota_t < hi), val_ref[0], 0.0)
        rl = rows_ref[0] - band * tm
        iota_r = jax.lax.broadcasted_iota(jnp.int32, (tm, ch), 0)
        oh = jnp.where(iota_r == rl, valm, 0.0)
        o_ref[...] += jnp.sum(oh, axis=1, keepdims=True)

    @pl.when(last_ref[sl] == 1)
    def _finalize():
        row_ids = band * tm + jax.lax.broadcasted_iota(
            jnp.int32, (tm, o_ref.shape[1]), 0)
        deg = o_ref[...] + (row_ids < num_nodes).astype(jnp.float32)
        o_ref[...] = jnp.where(deg > 0, jax.lax.rsqrt(deg), jnp.float32(0.0))


def _spmm_kernel(band_ref, lo_ref, hi_ref, first_ref, last_ref, chunk_ref,
                 cols_ref, rows_ref, val_ref, src_ref, d_ref, b_ref,
                 o_ref, g_ref, *, tm, ch, slots_half, unroll, final,
                 num_classes, c_pad):
    g = pl.program_id(0)
    s = pl.program_id(1)
    sl = g * slots_half + s
    band = band_ref[sl]
    lo = lo_ref[sl]
    hi = hi_ref[sl]

    @pl.when(first_ref[sl] == 1)
    def _init():
        o_ref[...] = src_ref[pl.ds(band * tm, tm), :]

    @pl.when(hi > lo)
    def _accumulate():
        _gather_rows(cols_ref, src_ref, g_ref, ch, unroll)
        iota_t = jax.lax.broadcasted_iota(jnp.int32, (1, ch), 1)
        valm = jnp.where((iota_t >= lo) & (iota_t < hi), val_ref[0], 0.0)
        rl = rows_ref[0] - band * tm
        iota_r = jax.lax.broadcasted_iota(jnp.int32, (tm, ch), 0)
        oh = jnp.where(iota_r == rl, valm, 0.0).astype(jnp.bfloat16)
        gb = g_ref[...].astype(jnp.bfloat16)
        o_ref[...] += jnp.dot(oh, gb, preferred_element_type=jnp.float32)

    if final:
        @pl.when(last_ref[sl] == 1)
        def _finalize():
            logits = o_ref[...] * d_ref[...] + b_ref[...]
            if num_classes < c_pad:
                col = jax.lax.broadcasted_iota(jnp.int32, logits.shape, 1)
                valid = col < num_classes
                logits = jnp.where(valid, logits, jnp.float32(-1e30))
                m = jnp.max(logits, axis=1, keepdims=True)
                e = jnp.where(valid, jnp.exp(logits - m), jnp.float32(0.0))
            else:
                m = jnp.max(logits, axis=1, keepdims=True)
                e = jnp.exp(logits - m)
            lse = jnp.log(jnp.sum(e, axis=1, keepdims=True)) + m
            o_ref[...] = logits - lse


def _slot_schedule(bstart, nb, nbh, core, nchunks, ch, slots_half):
    b0 = core * nbh
    nb_core = min(nbh, nb - b0)
    clo = bstart[b0:b0 + nb_core] // ch
    chi = jnp.maximum((bstart[b0 + 1:b0 + nb_core + 1] + ch - 1) // ch - 1, clo)
    span = chi - clo + 1
    scoff = jnp.concatenate([jnp.zeros(1, jnp.int32),
                             jnp.cumsum(span).astype(jnp.int32)])
    s = jnp.arange(slots_half, dtype=jnp.int32)
    bloc = jnp.clip(jnp.sum((scoff[None, :] <= s[:, None]).astype(jnp.int32),
                            axis=1) - 1, 0, nb_core - 1)
    band = (b0 + bloc).astype(jnp.int32)
    within = s - scoff[bloc]
    chunk = jnp.clip(clo[bloc] + within, 0, nchunks - 1).astype(jnp.int32)
    real = s < scoff[nb_core]
    base = chunk * ch
    lo = jnp.where(real, jnp.maximum(bstart[band], base) - base, 0)
    hi = jnp.where(real, jnp.minimum(bstart[band + 1], base + ch) - base, 0)
    hi = jnp.maximum(hi, lo)
    first = ((s == scoff[bloc]) & real).astype(jnp.int32)
    last = ((s == scoff[bloc + 1] - 1) & real).astype(jnp.int32)
    return band, lo.astype(jnp.int32), hi.astype(jnp.int32), first, last, chunk


def kernel(edge_index, x, weight, bias):
    N, F = x.shape
    C = weight.shape[1]
    tm = 512
    ch = 2048
    unroll = 128
    n_pad = _ru(N, 2 * tm)
    f_pad = _ru(F, 128)
    c_pad = _ru(C, 128)
    nb = n_pad // tm
    nbh = nb // 2

    src, dst = edge_index[0], edge_index[1]
    rows = jnp.concatenate([src, dst])
    cols = jnp.concatenate([dst, src])
    m = rows.shape[0]
    m_pad = _ru(m, ch)
    key = jnp.sort(rows * n_pad + cols)
    pad_key = jnp.full((m_pad - m,), n_pad * n_pad - 1, jnp.int32)
    key = jnp.concatenate([key, pad_key])
    rs = (key // n_pad).astype(jnp.int32)
    cs = (key % n_pad).astype(jnp.int32)
    is_real = jnp.arange(m_pad) < m
    uniq = jnp.concatenate([jnp.ones(1, jnp.bool_), key[1:] != key[:-1]])
    uniq = uniq & is_real
    val = uniq.astype(jnp.float32)

    nchunks = m_pad // ch
    slots_half = nchunks + nbh + 1
    q = (jnp.arange(nb + 1, dtype=jnp.int32) * tm)[:, None]
    bstart = jnp.sum((rs[None, :] < q).astype(jnp.int32), axis=1)
    sched = [jnp.concatenate([a, b]) for a, b in zip(
        _slot_schedule(bstart, nb, nbh, 0, nchunks, ch, slots_half),
        _slot_schedule(bstart, nb, nbh, 1, nchunks, ch, slots_half))]
    band_s, lo_s, hi_s, first_s, last_s, chunk_s = sched

    rows3 = rs.reshape(nchunks, 1, ch)
    cols3 = cs.reshape(nchunks, 1, ch)
    val3 = val.reshape(nchunks, 1, ch)

    def edge_map3(g, s, *pref):
        return (pref[5][g * slots_half + s], 0, 0)

    def out_map2(g, s, *pref):
        return (pref[0][g * slots_half + s], 0)

    d1 = pl.pallas_call(
        functools.partial(_deg_kernel, tm=tm, ch=ch, slots_half=slots_half,
                          num_nodes=N),
        out_shape=jax.ShapeDtypeStruct((n_pad, c_pad), jnp.float32),
        grid_spec=pltpu.PrefetchScalarGridSpec(
            num_scalar_prefetch=6,
            grid=(2, slots_half),
            in_specs=[
                pl.BlockSpec((1, 1, ch), edge_map3),
                pl.BlockSpec((1, 1, ch), edge_map3),
            ],
            out_specs=pl.BlockSpec((tm, c_pad), out_map2)),
        compiler_params=pltpu.CompilerParams(
            dimension_semantics=("parallel", "arbitrary")),
    )(band_s, lo_s, hi_s, first_s, last_s, chunk_s, rows3, val3)

    if N == n_pad and F == f_pad:
        x_p = x
    else:
        x_p = jnp.zeros((n_pad, f_pad), x.dtype).at[:N, :F].set(x)
    w_p = jnp.zeros((f_pad, c_pad), jnp.bfloat16).at[:F, :C].set(
        weight.astype(jnp.bfloat16))
    b_p = jnp.zeros((1, c_pad), jnp.float32).at[0, :C].set(
        bias.astype(jnp.float32))

    z0 = pl.pallas_call(
        _proj_kernel,
        out_shape=jax.ShapeDtypeStruct((n_pad, c_pad), jnp.float32),
        grid=(nb,),
        in_specs=[
            pl.BlockSpec((tm, f_pad), lambda i: (i, 0)),
            pl.BlockSpec((f_pad, c_pad), lambda i: (0, 0)),
            pl.BlockSpec((tm, c_pad), lambda i: (i, 0)),
        ],
        out_specs=pl.BlockSpec((tm, c_pad), lambda i: (i, 0)),
        compiler_params=pltpu.CompilerParams(
            dimension_semantics=("parallel",)),
    )(x_p, w_p, d1)

    def spmm(source, final):
        kern = functools.partial(
            _spmm_kernel, tm=tm, ch=ch, slots_half=slots_half, unroll=unroll,
            final=final, num_classes=C, c_pad=c_pad)
        smem = pltpu.MemorySpace.SMEM
        nargs = 6

        def edge_map(g, s, *pref):
            return (pref[5][g * slots_half + s], 0, 0)

        def out_map(g, s, *pref):
            return (pref[0][g * slots_half + s], 0)

        return pl.pallas_call(
            kern,
            out_shape=jax.ShapeDtypeStruct((n_pad, c_pad), jnp.float32),
            grid_spec=pltpu.PrefetchScalarGridSpec(
                num_scalar_prefetch=nargs,
                grid=(2, slots_half),
                in_specs=[
                    pl.BlockSpec((1, 1, ch), edge_map, memory_space=smem),
                    pl.BlockSpec((1, 1, ch), edge_map),
                    pl.BlockSpec((1, 1, ch), edge_map),
                    pl.BlockSpec((n_pad, c_pad), lambda g, s, *p: (0, 0)),
                    pl.BlockSpec((tm, c_pad), out_map),
                    pl.BlockSpec((1, c_pad), lambda g, s, *p: (0, 0)),
                ],
                out_specs=pl.BlockSpec((tm, c_pad), out_map),
                scratch_shapes=[pltpu.VMEM((ch, c_pad), jnp.float32)]),
            compiler_params=pltpu.CompilerParams(
                dimension_semantics=("parallel", "arbitrary"),
                vmem_limit_bytes=100 * 1024 * 1024),
        )(band_s, lo_s, hi_s, first_s, last_s, chunk_s,
          cols3, rows3, val3, source, d1, b_p)

    h = spmm(z0, final=False)
    h = h * d1 * d1

    out_p = spmm(h, final=True)

    return out_p[:N, :C]
